# Initial kernel scaffold; baseline (speedup 1.0000x reference)
import jax
import jax.numpy as jnp
from jax.experimental import pallas as pl


def kernel(x, edge_index, edge_attr, batch, W_embed, b_embed, Wf, bf, Ws, bs, gamma, beta, W1, b1, W2, b2, W3, b3):
    raise NotImplementedError("write your pallas kernel here")


# speedup vs baseline: 2.9519x; 2.9519x over previous
import functools

import jax
import jax.numpy as jnp
from jax import lax
from jax.experimental import pallas as pl
from jax.experimental.pallas import tpu as pltpu
from jax.experimental.pallas import tpu_sc as plsc

_N = 50000
_E = 800000
_ND = 92
_ED = 41
_H = 64
_HW = 128
_NG = 256

_W = 128
_EPAD = 819200
_NWIN = _EPAD // _W
_NPAD = 50176
_NRANGE = 4
_RROWS = _NPAD // _NRANGE
_ARES = _RROWS + 8
_SWB = _RROWS // 16
_ZCH = 56
_ZB = 64
_WS = 64
_BM = 2048
_NBLK = 2000


def _mesh():
    return plsc.VectorSubcoreMesh(core_axis_name="c", subcore_axis_name="s")


def _sc_gather(h, idx):
    @functools.partial(
        pl.kernel,
        out_type=jax.ShapeDtypeStruct((_EPAD, _HW), jnp.float32),
        mesh=_mesh(),
    )
    def gk(h_hbm, i_hbm, o_hbm):
        def body(iv, ov):
            pltpu.sync_copy(h_hbm.at[iv.at[0]], ov)

        pltpu.emit_pipeline(
            body,
            grid=(_NWIN,),
            in_specs=[pl.BlockSpec((1, _W), lambda i: (0, i))],
            out_specs=[pl.BlockSpec((_W, _HW), lambda i: (i, 0))],
            core_axis_name=("c", "s"),
            dimension_semantics=(pltpu.PARALLEL,),
        )(i_hbm, o_hbm)

    return gk(h, idx)


def _sc_scatter(msg, dst1, zrows):
    nwps = _EPAD // _WS // 16

    @functools.partial(
        pl.kernel,
        out_type=jax.ShapeDtypeStruct((_NPAD, _HW), jnp.float32),
        mesh=_mesh(),
        scratch_types=[pltpu.VMEM_SHARED((_ARES, _HW), jnp.float32)],
    )
    def sk(m_hbm, d_hbm, z_hbm, o_hbm, acc_sh):
        c = lax.axis_index("c")
        s = lax.axis_index("s")

        def fill_idx(izbuf, base):
            for k in range(4):
                izbuf.at[pl.ds(16 * k, 16)][...] = (
                    base + 16 * k
                    + lax.broadcasted_iota(jnp.int32, (16,), 0))

        def inner(wbuf, izbuf, mbuf0, mbuf1, ibuf0, ibuf1, jbuf0, jbuf1,
                  sm0, sm1, si0, si1):
            def one_pass(p):
                base = (c * 2 + p) * _RROWS

                pltpu.sync_copy(z_hbm, wbuf)

                @pl.loop(0, _SWB // _ZCH)
                def _(j):
                    fill_idx(izbuf, s * _SWB + j * _ZCH)
                    pltpu.sync_copy(wbuf, acc_sh.at[izbuf])

                @pl.when(s == 0)
                def _():
                    fill_idx(izbuf, _RROWS - _ZB + 8)
                    pltpu.sync_copy(wbuf, acc_sh.at[izbuf])

                plsc.subcore_barrier()

                bufs = ((mbuf0, ibuf0, jbuf0, sm0, si0),
                        (mbuf1, ibuf1, jbuf1, sm1, si1))

                def mcopy(j, k):
                    off = (s * nwps + j) * _WS
                    mb, ib, _, sm, si = bufs[k]
                    return (
                        pltpu.make_async_copy(
                            m_hbm.at[pl.ds(off, _WS), :], mb, sm),
                        pltpu.make_async_copy(
                            d_hbm.at[0, pl.ds(off, _WS)], ib, si),
                    )

                def scat(k):
                    mb, ib, jb, _, _ = bufs[k]
                    for q in range(4):
                        v = ib.at[pl.ds(16 * q, 16)][...]
                        local = v - base
                        ok = (local >= 0) & (local < _RROWS)
                        jb.at[pl.ds(16 * q, 16)][...] = jnp.where(
                            ok, local, _RROWS)
                    pltpu.sync_copy(mb, acc_sh.at[jb], add=True)

                for cp in mcopy(0, 0):
                    cp.start()

                @pl.loop(0, nwps, step=2)
                def _(j):
                    for cp in mcopy(j + 1, 1):
                        cp.start()
                    for cp in mcopy(j, 0):
                        cp.wait()
                    scat(0)

                    @pl.when(j + 2 < nwps)
                    def _():
                        for cp in mcopy(j + 2, 0):
                            cp.start()

                    for cp in mcopy(j + 1, 1):
                        cp.wait()
                    scat(1)

                plsc.subcore_barrier()

                @pl.loop(0, _SWB // _ZCH)
                def _(j):
                    row = s * _SWB + j * _ZCH
                    fill_idx(izbuf, row)
                    pltpu.sync_copy(acc_sh.at[izbuf], wbuf)
                    pltpu.sync_copy(wbuf.at[pl.ds(0, _ZCH), :],
                                    o_hbm.at[pl.ds(base + row, _ZCH), :])

                plsc.subcore_barrier()

            one_pass(0)
            one_pass(1)

        pl.run_scoped(inner,
                      pltpu.VMEM((_ZB, _HW), jnp.float32),
                      pltpu.VMEM((_ZB,), jnp.int32),
                      pltpu.VMEM((_WS, _HW), jnp.float32),
                      pltpu.VMEM((_WS, _HW), jnp.float32),
                      pltpu.VMEM((_WS,), jnp.int32),
                      pltpu.VMEM((_WS,), jnp.int32),
                      pltpu.VMEM((_WS,), jnp.int32),
                      pltpu.VMEM((_WS,), jnp.int32),
                      pltpu.SemaphoreType.DMA,
                      pltpu.SemaphoreType.DMA,
                      pltpu.SemaphoreType.DMA,
                      pltpu.SemaphoreType.DMA)

    return sk(msg, dst1, zrows)


def _tc_msg(hd, hs, ea, wd, wsr, we, bias):
    nb = _EPAD // _BM

    def mk(hd_ref, hs_ref, ea_ref, wd_ref, wsr_ref, we_ref, b_ref, o_ref):
        i = pl.program_id(0)
        acc = jnp.dot(hd_ref[...], wd_ref[...],
                      preferred_element_type=jnp.float32,
                      precision=lax.Precision.HIGHEST)
        acc += jnp.dot(hs_ref[...], wsr_ref[...],
                       preferred_element_type=jnp.float32,
                      precision=lax.Precision.HIGHEST)
        acc += jnp.dot(ea_ref[...], we_ref[...],
                       preferred_element_type=jnp.float32,
                      precision=lax.Precision.HIGHEST)
        acc += b_ref[...]
        x = acc[:, :_H]
        y = acc[:, _H:]
        gate = 1.0 / (1.0 + jnp.exp(-x))
        sp = jnp.maximum(y, 0.0) + jnp.log1p(jnp.exp(-jnp.abs(y)))
        msg = gate * sp
        rows = i * _BM + lax.broadcasted_iota(jnp.int32, (_BM, 1), 0)
        msg = jnp.where(rows < _E, msg, 0.0)
        o_ref[...] = jnp.concatenate(
            [msg, jnp.zeros((_BM, _HW - _H), jnp.float32)], axis=1)

    return pl.pallas_call(
        mk,
        grid=(nb,),
        in_specs=[pl.BlockSpec((_BM, _HW), lambda i: (i, 0)),
                  pl.BlockSpec((_BM, _HW), lambda i: (i, 0)),
                  pl.BlockSpec((_BM, 48), lambda i: (i, 0)),
                  pl.BlockSpec((_HW, 2 * _H), lambda i: (0, 0)),
                  pl.BlockSpec((_HW, 2 * _H), lambda i: (0, 0)),
                  pl.BlockSpec((48, 2 * _H), lambda i: (0, 0)),
                  pl.BlockSpec((1, 2 * _H), lambda i: (0, 0))],
        out_specs=pl.BlockSpec((_BM, _HW), lambda i: (i, 0)),
        out_shape=jax.ShapeDtypeStruct((_EPAD, _HW), jnp.float32),
    )(hd, hs, ea, wd, wsr, we, bias)


def _tc_bn_stats(agg):
    nb = _N // _NBLK

    def stk(a_ref, o_ref, s_s, q_s):
        p = pl.program_id(0)
        i = pl.program_id(1)

        @pl.when((p == 0) & (i == 0))
        def _():
            s_s[...] = jnp.zeros_like(s_s)
            q_s[...] = jnp.zeros_like(q_s)

        a = a_ref[...][:, :_H]

        @pl.when(p == 0)
        def _():
            s_s[...] += jnp.sum(a, axis=0, keepdims=True)

        @pl.when(p == 1)
        def _():
            d = a - s_s[...] / _N
            q_s[...] += jnp.sum(d * d, axis=0, keepdims=True)

        @pl.when((p == 1) & (i == nb - 1))
        def _():
            o_ref[...] = jnp.concatenate(
                [s_s[...] / _N, q_s[...] / _N,
                 jnp.zeros((6, _H), jnp.float32)], axis=0)

    return pl.pallas_call(
        stk,
        grid=(2, nb),
        in_specs=[pl.BlockSpec((_NBLK, _HW), lambda p, i: (i, 0))],
        out_specs=pl.BlockSpec((8, _H), lambda p, i: (0, 0)),
        out_shape=jax.ShapeDtypeStruct((8, _H), jnp.float32),
        scratch_shapes=[pltpu.VMEM((1, _H), jnp.float32),
                        pltpu.VMEM((1, _H), jnp.float32)],
    )(agg)


def _tc_bn_apply(agg, stats, h, gamma, beta):
    nb = _N // _NBLK

    def bk(a_ref, st_ref, h_ref, g_ref, b_ref, o_ref):
        a = a_ref[...][:, :_H]
        mu = st_ref[0:1, :]
        var = st_ref[1:2, :]
        hn = ((a - mu) * lax.rsqrt(var + 1e-5) * g_ref[...] + b_ref[...]
              + h_ref[...][:, :_H])
        hn = jnp.maximum(hn, 0.0)
        o_ref[...] = jnp.concatenate(
            [hn, jnp.zeros((_NBLK, _HW - _H), jnp.float32)], axis=1)

    return pl.pallas_call(
        bk,
        grid=(nb,),
        in_specs=[pl.BlockSpec((_NBLK, _HW), lambda i: (i, 0)),
                  pl.BlockSpec((8, _H), lambda i: (0, 0)),
                  pl.BlockSpec((_NBLK, _HW), lambda i: (i, 0)),
                  pl.BlockSpec((1, _H), lambda i: (0, 0)),
                  pl.BlockSpec((1, _H), lambda i: (0, 0))],
        out_specs=pl.BlockSpec((_NBLK, _HW), lambda i: (i, 0)),
        out_shape=jax.ShapeDtypeStruct((_N, _HW), jnp.float32),
    )(agg, stats, h, gamma, beta)


def _tc_pool_head(h, batch3, w1, b1, w2, b2, w3, b3):
    nb = _N // _NBLK

    def pk(h_ref, bt_ref, w1_ref, b1_ref, w2_ref, b2_ref, w3_ref, b3_ref,
           o_ref, acc_s, cnt_s):
        i = pl.program_id(0)

        @pl.when(i == 0)
        def _():
            acc_s[...] = jnp.zeros_like(acc_s)
            cnt_s[...] = jnp.zeros_like(cnt_s)

        b_row = bt_ref[0]
        iota2 = lax.broadcasted_iota(jnp.int32, (_NG, _NBLK), 0)
        oh = jnp.where(iota2 == b_row, 1.0, 0.0)
        acc_s[...] += lax.dot_general(
            oh, h_ref[...][:, :_H], (((1,), (0,)), ((), ())),
            preferred_element_type=jnp.float32,
                      precision=lax.Precision.HIGHEST)
        cnt_s[...] += lax.dot_general(
            oh, jnp.ones((_NBLK, 1), jnp.float32), (((1,), (0,)), ((), ())),
            preferred_element_type=jnp.float32,
                      precision=lax.Precision.HIGHEST)

        @pl.when(i == nb - 1)
        def _():
            pooled = acc_s[...] / jnp.maximum(cnt_s[...], 1.0)
            f = jnp.maximum(jnp.dot(pooled, w1_ref[...],
                                    preferred_element_type=jnp.float32,
                      precision=lax.Precision.HIGHEST)
                            + b1_ref[...], 0.0)
            f = jnp.maximum(jnp.dot(f, w2_ref[...],
                                    preferred_element_type=jnp.float32,
                      precision=lax.Precision.HIGHEST)
                            + b2_ref[...], 0.0)
            res = lax.dot_general(w3_ref[...], f, (((0,), (1,)), ((), ())),
                                  preferred_element_type=jnp.float32,
                      precision=lax.Precision.HIGHEST)
            o_ref[...] = res + b3_ref[...]

    return pl.pallas_call(
        pk,
        grid=(nb,),
        in_specs=[pl.BlockSpec((_NBLK, _HW), lambda i: (i, 0)),
                  pl.BlockSpec((1, 1, _NBLK), lambda i: (i, 0, 0)),
                  pl.BlockSpec((_H, _H // 2), lambda i: (0, 0)),
                  pl.BlockSpec((1, _H // 2), lambda i: (0, 0)),
                  pl.BlockSpec((_H // 2, _H // 2), lambda i: (0, 0)),
                  pl.BlockSpec((1, _H // 2), lambda i: (0, 0)),
                  pl.BlockSpec((_H // 2, 1), lambda i: (0, 0)),
                  pl.BlockSpec((1, 1), lambda i: (0, 0))],
        out_specs=pl.BlockSpec((1, _NG), lambda i: (0, 0)),
        out_shape=jax.ShapeDtypeStruct((1, _NG), jnp.float32),
        scratch_shapes=[pltpu.VMEM((_NG, _H), jnp.float32),
                        pltpu.VMEM((_NG, 1), jnp.float32)],
    )(h, batch3, w1, b1, w2, b2, w3, b3)


def _tc_embed(xp, wm, bm):
    def ek(x_ref, w_ref, b_ref, o_ref):
        e = jnp.dot(x_ref[...], w_ref[...],
                    preferred_element_type=jnp.float32,
                      precision=lax.Precision.HIGHEST) + b_ref[...]
        o_ref[...] = jnp.concatenate(
            [e, jnp.zeros((_NBLK, _HW - _H), jnp.float32)], axis=1)

    return pl.pallas_call(
        ek,
        grid=(_N // _NBLK,),
        in_specs=[pl.BlockSpec((_NBLK, 96), lambda i: (i, 0)),
                  pl.BlockSpec((96, _H), lambda i: (0, 0)),
                  pl.BlockSpec((1, _H), lambda i: (0, 0))],
        out_specs=pl.BlockSpec((_NBLK, _HW), lambda i: (i, 0)),
        out_shape=jax.ShapeDtypeStruct((_N, _HW), jnp.float32),
    )(xp, wm, bm)


def kernel(x, edge_index, edge_attr, batch, W_embed, b_embed, Wf, bf, Ws, bs,
           gamma, beta, W1, b1, W2, b2, W3, b3):
    src = edge_index[0]
    dst = edge_index[1]
    npad = _EPAD - _E
    padidx = (jnp.arange(npad, dtype=jnp.int32) * 997) % _N
    dst_p = jnp.concatenate([dst, padidx]).reshape(1, _EPAD)
    src_p = jnp.concatenate([src, padidx]).reshape(1, _EPAD)
    ea_p = jnp.zeros((_EPAD, 48), jnp.float32).at[:_E, :_ED].set(edge_attr)
    xp = jnp.pad(x, ((0, 0), (0, 96 - _ND)))
    wm = jnp.pad(W_embed, ((0, 96 - _ND), (0, 0)))
    zrows = jnp.zeros((_ZB, _HW), jnp.float32)

    h = _tc_embed(xp, wm, b_embed.reshape(1, _H))

    for i in range(3):
        wd = jnp.pad(
            jnp.concatenate([Wf[i, :_H], Ws[i, :_H]], axis=1),
            ((0, _HW - _H), (0, 0)))
        wsr = jnp.pad(
            jnp.concatenate([Wf[i, _H:2 * _H], Ws[i, _H:2 * _H]], axis=1),
            ((0, _HW - _H), (0, 0)))
        we = jnp.pad(jnp.concatenate([Wf[i, 2 * _H:], Ws[i, 2 * _H:]], axis=1),
                     ((0, 7), (0, 0)))
        bias = jnp.concatenate([bf[i], bs[i]]).reshape(1, 2 * _H)

        hd = _sc_gather(h, dst_p)
        hs = _sc_gather(h, src_p)
        msg = _tc_msg(hd, hs, ea_p, wd, wsr, we, bias)
        agg = _sc_scatter(msg, dst_p, zrows)
        stats = _tc_bn_stats(agg)
        h = _tc_bn_apply(agg, stats, h, gamma[i].reshape(1, _H),
                         beta[i].reshape(1, _H))

    out = _tc_pool_head(h, batch.reshape(_N // _NBLK, 1, _NBLK),
                        W1, b1.reshape(1, _H // 2), W2, b2.reshape(1, _H // 2),
                        W3, b3.reshape(1, 1))
    return out.reshape(_NG)
